# baseline (speedup 1.0000x reference)
import jax
import jax.numpy as jnp
from jax import lax
from jax.experimental import pallas as pl
from jax.experimental.pallas import tpu as pltpu
from jax.experimental.pallas import tpu_sc as plsc

DIM = 1024
INTER = 1024
E = 8
K = 2
T = 2048
TK = T * K
TILE = 256
LANES = 128
GTILE = 128
MAX_TILES = 39
PAD_ROWS = MAX_TILES * GTILE


def _dot_t(a, b):
    return jax.lax.dot_general(
        a, b, (((1,), (1,)), ((), ())), preferred_element_type=jnp.float32
    )


def _router_kernel(x_ref, gw_ref, gb_ref, out_ref, cnt_ref, carry):
    t = pl.program_id(0)

    @pl.when(t == 0)
    def _():
        carry[...] = jnp.zeros_like(carry)

    x = x_ref[...]
    logits = _dot_t(x, gw_ref[...]) + gb_ref[0:1, :]
    lane = jax.lax.broadcasted_iota(jnp.int32, logits.shape, 1)
    probs = jnp.where(lane < E, jax.nn.sigmoid(logits), -1.0)
    i1 = jnp.argmax(probs, axis=-1)
    oh1 = (lane == i1[:, None]).astype(jnp.float32)
    m1 = jnp.max(probs, axis=-1, keepdims=True)
    probs2 = jnp.where(oh1 > 0, -1.0, probs)
    i2 = jnp.argmax(probs2, axis=-1)
    oh2 = (lane == i2[:, None]).astype(jnp.float32)
    m2 = jnp.max(probs2, axis=-1, keepdims=True)
    s = m1 + m2 + 1e-8
    w0 = m1 / s
    w1 = m2 / s
    ra = jax.lax.broadcasted_iota(jnp.int32, (TILE, TILE), 0)
    ca = jax.lax.broadcasted_iota(jnp.int32, (TILE, TILE), 1)
    tril = (ra > ca).astype(jnp.float32)
    pre1 = jax.lax.dot_general(
        tril, oh1, (((1,), (0,)), ((), ())), preferred_element_type=jnp.float32
    )
    pre2 = jax.lax.dot_general(
        tril, oh2, (((1,), (0,)), ((), ())), preferred_element_type=jnp.float32
    )
    c = carry[0:1, :]
    cnt1 = jnp.sum(oh1, axis=0, keepdims=True)
    rank0 = jnp.sum((pre1 + c) * oh1, axis=1, keepdims=True)
    rank1 = jnp.sum((pre2 + c + cnt1) * oh2, axis=1, keepdims=True)
    new_c = c + cnt1 + jnp.sum(oh2, axis=0, keepdims=True)
    carry[0:1, :] = new_c
    cnt_ref[...] = jnp.broadcast_to(new_c, cnt_ref.shape)
    out = (
        jnp.where(lane == 0, i1[:, None].astype(jnp.float32), 0.0)
        + jnp.where(lane == 1, i2[:, None].astype(jnp.float32), 0.0)
        + jnp.where(lane == 2, w0, 0.0)
        + jnp.where(lane == 3, w1, 0.0)
        + jnp.where(lane == 4, rank0, 0.0)
        + jnp.where(lane == 5, rank1, 0.0)
    )
    out_ref[...] = out.T


NW = 32
TPW = T // NW


def _dispatch_sc_kernel(
    xt_hbm, pos0_hbm, pos1_hbm, xs_hbm, idx0_v, idx1_v, rows_v, sem0, sem1
):
    wid = lax.axis_index("c") * 16 + lax.axis_index("s")
    base = wid * TPW
    pltpu.sync_copy(pos0_hbm.at[pl.ds(base, TPW)], idx0_v)
    pltpu.sync_copy(pos1_hbm.at[pl.ds(base, TPW)], idx1_v)
    pltpu.sync_copy(xt_hbm.at[pl.ds(base, TPW)], rows_v)
    c0 = pltpu.async_copy(rows_v, xs_hbm.at[idx0_v], sem0)
    c1 = pltpu.async_copy(rows_v, xs_hbm.at[idx1_v], sem1)
    c0.wait()
    c1.wait()


def _group_kernel(meta_ref, xs_ref, w1_ref, w3_ref, w2_ref, o_ref):
    j = pl.program_id(0)
    n_valid = meta_ref[MAX_TILES]

    @pl.when(j < n_valid)
    def _():
        x = xs_ref[...]
        h1 = _dot_t(x, w1_ref[0])
        h3 = _dot_t(x, w3_ref[0])
        h = (h1 * jax.nn.sigmoid(h1)) * h3
        o_ref[...] = _dot_t(h, w2_ref[0])


def _shared_kernel(x_ref, f1_ref, f2_ref, f3_ref, o_ref):
    x = x_ref[...]
    h1 = _dot_t(x, f1_ref[...])
    h3 = _dot_t(x, f2_ref[...])
    h = (h1 * jax.nn.sigmoid(h1)) * h3
    o_ref[...] = _dot_t(h, f3_ref[...])


def _combine_kernel(z_ref, g0_ref, g1_ref, r_ref, o_ref):
    wcols = r_ref[2:4, :].T
    o_ref[...] = (
        z_ref[...]
        + wcols[:, 0:1] * g0_ref[...]
        + wcols[:, 1:2] * g1_ref[...]
    )


def kernel(x, gate_w, gate_b, w1, w2, w3, fc1, fc2, fc3):
    orig_shape = x.shape
    xt = x.reshape(T, DIM)

    gw_pad = jnp.zeros((LANES, DIM), jnp.float32).at[:E].set(gate_w)
    gb_pad = jnp.zeros((8, LANES), jnp.float32).at[:, :E].set(
        jnp.broadcast_to(gate_b, (8, E))
    )

    rout, cnt = pl.pallas_call(
        _router_kernel,
        grid=(T // TILE,),
        in_specs=[
            pl.BlockSpec((TILE, DIM), lambda t: (t, 0)),
            pl.BlockSpec((LANES, DIM), lambda t: (0, 0)),
            pl.BlockSpec((8, LANES), lambda t: (0, 0)),
        ],
        out_specs=[
            pl.BlockSpec((LANES, TILE), lambda t: (0, t)),
            pl.BlockSpec((8, LANES), lambda t: (0, 0)),
        ],
        out_shape=[
            jax.ShapeDtypeStruct((LANES, T), jnp.float32),
            jax.ShapeDtypeStruct((8, LANES), jnp.float32),
        ],
        scratch_shapes=[pltpu.VMEM((8, LANES), jnp.float32)],
    )(xt, gw_pad, gb_pad)

    idx0 = rout[0].astype(jnp.int32)
    idx1 = rout[1].astype(jnp.int32)
    counts = cnt[0, :E].astype(jnp.int32)
    padded = ((counts + GTILE - 1) // GTILE) * GTILE
    ends = jnp.cumsum(padded)
    pstart = ends - padded
    pos0 = pstart[idx0] + rout[4].astype(jnp.int32)
    pos1 = pstart[idx1] + rout[5].astype(jnp.int32)
    tile_expert = jnp.clip(
        jnp.searchsorted(ends, jnp.arange(MAX_TILES) * GTILE, side="right"),
        0,
        E - 1,
    ).astype(jnp.int32)
    n_tiles = (ends[-1] // GTILE).astype(jnp.int32)
    meta = jnp.concatenate([tile_expert, n_tiles[None]])

    xs = pl.kernel(
        _dispatch_sc_kernel,
        mesh=plsc.VectorSubcoreMesh(core_axis_name="c", subcore_axis_name="s"),
        out_type=jax.ShapeDtypeStruct((PAD_ROWS, DIM), jnp.float32),
        scratch_types=[
            pltpu.VMEM((TPW,), jnp.int32),
            pltpu.VMEM((TPW,), jnp.int32),
            pltpu.VMEM((TPW, DIM), jnp.float32),
            pltpu.SemaphoreType.DMA,
            pltpu.SemaphoreType.DMA,
        ],
    )(xt, pos0, pos1)

    eo = pl.pallas_call(
        _group_kernel,
        grid_spec=pltpu.PrefetchScalarGridSpec(
            num_scalar_prefetch=1,
            grid=(MAX_TILES,),
            in_specs=[
                pl.BlockSpec((GTILE, DIM), lambda j, te: (j, 0)),
                pl.BlockSpec((1, INTER, DIM), lambda j, te: (te[j], 0, 0)),
                pl.BlockSpec((1, INTER, DIM), lambda j, te: (te[j], 0, 0)),
                pl.BlockSpec((1, DIM, INTER), lambda j, te: (te[j], 0, 0)),
            ],
            out_specs=pl.BlockSpec((GTILE, DIM), lambda j, te: (j, 0)),
        ),
        out_shape=jax.ShapeDtypeStruct((PAD_ROWS, DIM), jnp.float32),
    )(meta, xs, w1, w3, w2)

    z = pl.pallas_call(
        _shared_kernel,
        grid=(T // TILE,),
        in_specs=[
            pl.BlockSpec((TILE, DIM), lambda t: (t, 0)),
            pl.BlockSpec((INTER, DIM), lambda t: (0, 0)),
            pl.BlockSpec((INTER, DIM), lambda t: (0, 0)),
            pl.BlockSpec((DIM, INTER), lambda t: (0, 0)),
        ],
        out_specs=pl.BlockSpec((TILE, DIM), lambda t: (t, 0)),
        out_shape=jax.ShapeDtypeStruct((T, DIM), jnp.float32),
    )(xt, fc1, fc2, fc3)

    g0 = eo[pos0]
    g1 = eo[pos1]

    out = pl.pallas_call(
        _combine_kernel,
        grid=(T // TILE,),
        in_specs=[
            pl.BlockSpec((TILE, DIM), lambda t: (t, 0)),
            pl.BlockSpec((TILE, DIM), lambda t: (t, 0)),
            pl.BlockSpec((TILE, DIM), lambda t: (t, 0)),
            pl.BlockSpec((8, TILE), lambda t: (0, t)),
        ],
        out_specs=pl.BlockSpec((TILE, DIM), lambda t: (t, 0)),
        out_shape=jax.ShapeDtypeStruct((T, DIM), jnp.float32),
    )(z, g0, g1, rout)

    return out.reshape(orig_shape)


# speedup vs baseline: 1.2331x; 1.2331x over previous
import jax
import jax.numpy as jnp
from jax import lax
from jax.experimental import pallas as pl
from jax.experimental.pallas import tpu as pltpu
from jax.experimental.pallas import tpu_sc as plsc

DIM = 1024
INTER = 1024
E = 8
K = 2
T = 2048
TK = T * K
TILE = 256
LANES = 128
GTILE = 256
MAX_TILES = 23
PAD_ROWS = MAX_TILES * GTILE


def _dot_t(a, b):
    return jax.lax.dot_general(
        a, b, (((1,), (1,)), ((), ())), preferred_element_type=jnp.float32
    )


def _router_kernel(x_ref, gw_ref, gb_ref, out_ref, cnt_ref, carry):
    t = pl.program_id(0)

    @pl.when(t == 0)
    def _():
        carry[...] = jnp.zeros_like(carry)

    x = x_ref[...]
    logits = _dot_t(x, gw_ref[...]) + gb_ref[0:1, :]
    lane = jax.lax.broadcasted_iota(jnp.int32, logits.shape, 1)
    probs = jnp.where(lane < E, jax.nn.sigmoid(logits), -1.0)
    i1 = jnp.argmax(probs, axis=-1)
    oh1 = (lane == i1[:, None]).astype(jnp.float32)
    m1 = jnp.max(probs, axis=-1, keepdims=True)
    probs2 = jnp.where(oh1 > 0, -1.0, probs)
    i2 = jnp.argmax(probs2, axis=-1)
    oh2 = (lane == i2[:, None]).astype(jnp.float32)
    m2 = jnp.max(probs2, axis=-1, keepdims=True)
    s = m1 + m2 + 1e-8
    w0 = m1 / s
    w1 = m2 / s
    ra = jax.lax.broadcasted_iota(jnp.int32, (TILE, TILE), 0)
    ca = jax.lax.broadcasted_iota(jnp.int32, (TILE, TILE), 1)
    tril = (ra > ca).astype(jnp.float32)
    pre1 = jax.lax.dot_general(
        tril, oh1, (((1,), (0,)), ((), ())), preferred_element_type=jnp.float32
    )
    pre2 = jax.lax.dot_general(
        tril, oh2, (((1,), (0,)), ((), ())), preferred_element_type=jnp.float32
    )
    c = carry[0:1, :]
    cnt1 = jnp.sum(oh1, axis=0, keepdims=True)
    rank0 = jnp.sum((pre1 + c) * oh1, axis=1, keepdims=True)
    rank1 = jnp.sum((pre2 + c + cnt1) * oh2, axis=1, keepdims=True)
    new_c = c + cnt1 + jnp.sum(oh2, axis=0, keepdims=True)
    carry[0:1, :] = new_c
    cnt_ref[...] = jnp.broadcast_to(new_c, cnt_ref.shape)
    out = (
        jnp.where(lane == 0, i1[:, None].astype(jnp.float32), 0.0)
        + jnp.where(lane == 1, i2[:, None].astype(jnp.float32), 0.0)
        + jnp.where(lane == 2, w0, 0.0)
        + jnp.where(lane == 3, w1, 0.0)
        + jnp.where(lane == 4, rank0, 0.0)
        + jnp.where(lane == 5, rank1, 0.0)
    )
    out_ref[...] = out.T


NW = 32
TPW = T // NW


def _dispatch_sc_kernel(
    xt_hbm, pos0_hbm, pos1_hbm, xs_hbm, idx0_v, idx1_v, rows_v, sem0, sem1
):
    wid = lax.axis_index("c") * 16 + lax.axis_index("s")
    base = wid * TPW
    pltpu.sync_copy(pos0_hbm.at[pl.ds(base, TPW)], idx0_v)
    pltpu.sync_copy(pos1_hbm.at[pl.ds(base, TPW)], idx1_v)
    pltpu.sync_copy(xt_hbm.at[pl.ds(base, TPW)], rows_v)
    c0 = pltpu.async_copy(rows_v, xs_hbm.at[idx0_v], sem0)
    c1 = pltpu.async_copy(rows_v, xs_hbm.at[idx1_v], sem1)
    c0.wait()
    c1.wait()


def _group_kernel(meta_ref, xs_ref, w1_ref, w3_ref, w2_ref, o_ref):
    j = pl.program_id(0)
    n_valid = meta_ref[MAX_TILES]

    @pl.when(j < n_valid)
    def _():
        x = xs_ref[...]
        h1 = _dot_t(x, w1_ref[0])
        h3 = _dot_t(x, w3_ref[0])
        h = (h1 * jax.nn.sigmoid(h1)) * h3
        o_ref[...] = _dot_t(h, w2_ref[0])


def _shared_kernel(x_ref, f1_ref, f2_ref, f3_ref, o_ref):
    x = x_ref[...]
    h1 = _dot_t(x, f1_ref[...])
    h3 = _dot_t(x, f2_ref[...])
    h = (h1 * jax.nn.sigmoid(h1)) * h3
    o_ref[...] = _dot_t(h, f3_ref[...])


def _combine_kernel(z_ref, g0_ref, g1_ref, r_ref, o_ref):
    wcols = r_ref[2:4, :].T
    o_ref[...] = (
        z_ref[...]
        + wcols[:, 0:1] * g0_ref[...]
        + wcols[:, 1:2] * g1_ref[...]
    )


def kernel(x, gate_w, gate_b, w1, w2, w3, fc1, fc2, fc3):
    orig_shape = x.shape
    xt = x.reshape(T, DIM)

    gw_pad = jnp.zeros((LANES, DIM), jnp.float32).at[:E].set(gate_w)
    gb_pad = jnp.zeros((8, LANES), jnp.float32).at[:, :E].set(
        jnp.broadcast_to(gate_b, (8, E))
    )

    rout, cnt = pl.pallas_call(
        _router_kernel,
        grid=(T // TILE,),
        in_specs=[
            pl.BlockSpec((TILE, DIM), lambda t: (t, 0)),
            pl.BlockSpec((LANES, DIM), lambda t: (0, 0)),
            pl.BlockSpec((8, LANES), lambda t: (0, 0)),
        ],
        out_specs=[
            pl.BlockSpec((LANES, TILE), lambda t: (0, t)),
            pl.BlockSpec((8, LANES), lambda t: (0, 0)),
        ],
        out_shape=[
            jax.ShapeDtypeStruct((LANES, T), jnp.float32),
            jax.ShapeDtypeStruct((8, LANES), jnp.float32),
        ],
        scratch_shapes=[pltpu.VMEM((8, LANES), jnp.float32)],
    )(xt, gw_pad, gb_pad)

    idx0 = rout[0].astype(jnp.int32)
    idx1 = rout[1].astype(jnp.int32)
    counts = cnt[0, :E].astype(jnp.int32)
    padded = ((counts + GTILE - 1) // GTILE) * GTILE
    ends = jnp.cumsum(padded)
    pstart = ends - padded
    pos0 = pstart[idx0] + rout[4].astype(jnp.int32)
    pos1 = pstart[idx1] + rout[5].astype(jnp.int32)
    tile_expert = jnp.clip(
        jnp.searchsorted(ends, jnp.arange(MAX_TILES) * GTILE, side="right"),
        0,
        E - 1,
    ).astype(jnp.int32)
    n_tiles = (ends[-1] // GTILE).astype(jnp.int32)
    meta = jnp.concatenate([tile_expert, n_tiles[None]])

    xs = pl.kernel(
        _dispatch_sc_kernel,
        mesh=plsc.VectorSubcoreMesh(core_axis_name="c", subcore_axis_name="s"),
        out_type=jax.ShapeDtypeStruct((PAD_ROWS, DIM), jnp.float32),
        scratch_types=[
            pltpu.VMEM((TPW,), jnp.int32),
            pltpu.VMEM((TPW,), jnp.int32),
            pltpu.VMEM((TPW, DIM), jnp.float32),
            pltpu.SemaphoreType.DMA,
            pltpu.SemaphoreType.DMA,
        ],
    )(xt, pos0, pos1)

    eo = pl.pallas_call(
        _group_kernel,
        grid_spec=pltpu.PrefetchScalarGridSpec(
            num_scalar_prefetch=1,
            grid=(MAX_TILES,),
            in_specs=[
                pl.BlockSpec((GTILE, DIM), lambda j, te: (j, 0)),
                pl.BlockSpec((1, INTER, DIM), lambda j, te: (te[j], 0, 0)),
                pl.BlockSpec((1, INTER, DIM), lambda j, te: (te[j], 0, 0)),
                pl.BlockSpec((1, DIM, INTER), lambda j, te: (te[j], 0, 0)),
            ],
            out_specs=pl.BlockSpec((GTILE, DIM), lambda j, te: (j, 0)),
        ),
        out_shape=jax.ShapeDtypeStruct((PAD_ROWS, DIM), jnp.float32),
    )(meta, xs, w1, w3, w2)

    z = pl.pallas_call(
        _shared_kernel,
        grid=(T // TILE,),
        in_specs=[
            pl.BlockSpec((TILE, DIM), lambda t: (t, 0)),
            pl.BlockSpec((INTER, DIM), lambda t: (0, 0)),
            pl.BlockSpec((INTER, DIM), lambda t: (0, 0)),
            pl.BlockSpec((DIM, INTER), lambda t: (0, 0)),
        ],
        out_specs=pl.BlockSpec((TILE, DIM), lambda t: (t, 0)),
        out_shape=jax.ShapeDtypeStruct((T, DIM), jnp.float32),
    )(xt, fc1, fc2, fc3)

    g0 = eo[pos0]
    g1 = eo[pos1]

    out = pl.pallas_call(
        _combine_kernel,
        grid=(T // TILE,),
        in_specs=[
            pl.BlockSpec((TILE, DIM), lambda t: (t, 0)),
            pl.BlockSpec((TILE, DIM), lambda t: (t, 0)),
            pl.BlockSpec((TILE, DIM), lambda t: (t, 0)),
            pl.BlockSpec((8, TILE), lambda t: (0, t)),
        ],
        out_specs=pl.BlockSpec((TILE, DIM), lambda t: (t, 0)),
        out_shape=jax.ShapeDtypeStruct((T, DIM), jnp.float32),
    )(z, g0, g1, rout)

    return out.reshape(orig_shape)


# speedup vs baseline: 1.2612x; 1.0228x over previous
import jax
import jax.numpy as jnp
from jax import lax
from jax.experimental import pallas as pl
from jax.experimental.pallas import tpu as pltpu
from jax.experimental.pallas import tpu_sc as plsc

DIM = 1024
INTER = 1024
E = 8
K = 2
T = 2048
TK = T * K
TILE = 256
LANES = 128
GTILE = 256
MAX_TILES = 23
PAD_ROWS = MAX_TILES * GTILE


def _dot_t(a, b):
    return jax.lax.dot_general(
        a, b, (((1,), (1,)), ((), ())), preferred_element_type=jnp.float32
    )


def _router_kernel(x_ref, gw_ref, gb_ref, out_ref, cnt_ref, carry):
    t = pl.program_id(0)

    @pl.when(t == 0)
    def _():
        carry[...] = jnp.zeros_like(carry)

    x = x_ref[...]
    logits = _dot_t(x, gw_ref[...]) + gb_ref[0:1, :]
    lane = jax.lax.broadcasted_iota(jnp.int32, logits.shape, 1)
    probs = jax.nn.sigmoid(logits)
    i1 = jnp.argmax(probs, axis=-1)
    oh1 = (lane == i1[:, None]).astype(jnp.float32)
    m1 = jnp.max(probs, axis=-1, keepdims=True)
    probs2 = jnp.where(oh1 > 0, -1.0, probs)
    i2 = jnp.argmax(probs2, axis=-1)
    oh2 = (lane == i2[:, None]).astype(jnp.float32)
    m2 = jnp.max(probs2, axis=-1, keepdims=True)
    s = m1 + m2 + 1e-8
    w0 = m1 / s
    w1 = m2 / s
    ra = jax.lax.broadcasted_iota(jnp.int32, (TILE, TILE), 0)
    ca = jax.lax.broadcasted_iota(jnp.int32, (TILE, TILE), 1)
    tril = (ra > ca).astype(jnp.float32)
    pre1 = jax.lax.dot_general(
        tril, oh1, (((1,), (0,)), ((), ())), preferred_element_type=jnp.float32
    )
    pre2 = jax.lax.dot_general(
        tril, oh2, (((1,), (0,)), ((), ())), preferred_element_type=jnp.float32
    )
    c = carry[0:1, :]
    cnt1 = jnp.sum(oh1, axis=0, keepdims=True)
    rank0 = jnp.sum((pre1 + c) * oh1, axis=1, keepdims=True)
    rank1 = jnp.sum((pre2 + c + cnt1) * oh2, axis=1, keepdims=True)
    new_c = c + cnt1 + jnp.sum(oh2, axis=0, keepdims=True)
    carry[0:1, :] = new_c
    cnt_ref[...] = jnp.broadcast_to(new_c, cnt_ref.shape)
    out = (
        jnp.where(lane == 0, i1[:, None].astype(jnp.float32), 0.0)
        + jnp.where(lane == 1, i2[:, None].astype(jnp.float32), 0.0)
        + jnp.where(lane == 2, w0, 0.0)
        + jnp.where(lane == 3, w1, 0.0)
        + jnp.where(lane == 4, rank0, 0.0)
        + jnp.where(lane == 5, rank1, 0.0)
    )
    out_ref[...] = out.T


NW = 32
TPW = T // NW


def _dispatch_sc_kernel(
    xt_hbm, pos0_hbm, pos1_hbm, xs_hbm, idx0_v, idx1_v, rows_v, sem0, sem1
):
    wid = lax.axis_index("c") * 16 + lax.axis_index("s")
    base = wid * TPW
    pltpu.sync_copy(pos0_hbm.at[pl.ds(base, TPW)], idx0_v)
    pltpu.sync_copy(pos1_hbm.at[pl.ds(base, TPW)], idx1_v)
    pltpu.sync_copy(xt_hbm.at[pl.ds(base, TPW)], rows_v)
    c0 = pltpu.async_copy(rows_v, xs_hbm.at[idx0_v], sem0)
    c1 = pltpu.async_copy(rows_v, xs_hbm.at[idx1_v], sem1)
    c0.wait()
    c1.wait()


def _group_kernel(meta_ref, xs_ref, w1_ref, w3_ref, w2_ref, o_ref):
    j = pl.program_id(0)
    n_valid = meta_ref[MAX_TILES]

    @pl.when(j < n_valid)
    def _():
        x = xs_ref[...]
        h1 = _dot_t(x, w1_ref[0])
        h3 = _dot_t(x, w3_ref[0])
        h = (h1 * jax.nn.sigmoid(h1)) * h3
        o_ref[...] = _dot_t(h, w2_ref[0])


def _shared_kernel(x_ref, f1_ref, f2_ref, f3_ref, o_ref):
    x = x_ref[...]
    h1 = _dot_t(x, f1_ref[...])
    h3 = _dot_t(x, f2_ref[...])
    h = (h1 * jax.nn.sigmoid(h1)) * h3
    o_ref[...] = _dot_t(h, f3_ref[...])


def _combine_kernel(z_ref, g0_ref, g1_ref, r_ref, o_ref):
    wcols = r_ref[2:4, :].T
    o_ref[...] = (
        z_ref[...]
        + wcols[:, 0:1] * g0_ref[...]
        + wcols[:, 1:2] * g1_ref[...]
    )


def kernel(x, gate_w, gate_b, w1, w2, w3, fc1, fc2, fc3):
    orig_shape = x.shape
    xt = x.reshape(T, DIM)

    gb_2d = jnp.broadcast_to(gate_b, (8, E))

    rout, cnt = pl.pallas_call(
        _router_kernel,
        grid=(T // TILE,),
        in_specs=[
            pl.BlockSpec((TILE, DIM), lambda t: (t, 0)),
            pl.BlockSpec((E, DIM), lambda t: (0, 0)),
            pl.BlockSpec((8, E), lambda t: (0, 0)),
        ],
        out_specs=[
            pl.BlockSpec((8, TILE), lambda t: (0, t)),
            pl.BlockSpec((8, E), lambda t: (0, 0)),
        ],
        out_shape=[
            jax.ShapeDtypeStruct((8, T), jnp.float32),
            jax.ShapeDtypeStruct((8, E), jnp.float32),
        ],
        scratch_shapes=[pltpu.VMEM((8, E), jnp.float32)],
    )(xt, gate_w, gb_2d)

    idx0 = rout[0].astype(jnp.int32)
    idx1 = rout[1].astype(jnp.int32)
    counts = cnt[0, :E].astype(jnp.int32)
    padded = ((counts + GTILE - 1) // GTILE) * GTILE
    ends = jnp.cumsum(padded)
    pstart = ends - padded
    pos0 = pstart[idx0] + rout[4].astype(jnp.int32)
    pos1 = pstart[idx1] + rout[5].astype(jnp.int32)
    tile_expert = jnp.clip(
        jnp.searchsorted(ends, jnp.arange(MAX_TILES) * GTILE, side="right"),
        0,
        E - 1,
    ).astype(jnp.int32)
    n_tiles = (ends[-1] // GTILE).astype(jnp.int32)
    meta = jnp.concatenate([tile_expert, n_tiles[None]])

    xs = pl.kernel(
        _dispatch_sc_kernel,
        mesh=plsc.VectorSubcoreMesh(core_axis_name="c", subcore_axis_name="s"),
        out_type=jax.ShapeDtypeStruct((PAD_ROWS, DIM), jnp.float32),
        scratch_types=[
            pltpu.VMEM((TPW,), jnp.int32),
            pltpu.VMEM((TPW,), jnp.int32),
            pltpu.VMEM((TPW, DIM), jnp.float32),
            pltpu.SemaphoreType.DMA,
            pltpu.SemaphoreType.DMA,
        ],
    )(xt, pos0, pos1)

    eo = pl.pallas_call(
        _group_kernel,
        grid_spec=pltpu.PrefetchScalarGridSpec(
            num_scalar_prefetch=1,
            grid=(MAX_TILES,),
            in_specs=[
                pl.BlockSpec((GTILE, DIM), lambda j, te: (j, 0)),
                pl.BlockSpec((1, INTER, DIM), lambda j, te: (te[j], 0, 0)),
                pl.BlockSpec((1, INTER, DIM), lambda j, te: (te[j], 0, 0)),
                pl.BlockSpec((1, DIM, INTER), lambda j, te: (te[j], 0, 0)),
            ],
            out_specs=pl.BlockSpec((GTILE, DIM), lambda j, te: (j, 0)),
        ),
        out_shape=jax.ShapeDtypeStruct((PAD_ROWS, DIM), jnp.float32),
    )(meta, xs, w1, w3, w2)

    z = pl.pallas_call(
        _shared_kernel,
        grid=(T // TILE,),
        in_specs=[
            pl.BlockSpec((TILE, DIM), lambda t: (t, 0)),
            pl.BlockSpec((INTER, DIM), lambda t: (0, 0)),
            pl.BlockSpec((INTER, DIM), lambda t: (0, 0)),
            pl.BlockSpec((DIM, INTER), lambda t: (0, 0)),
        ],
        out_specs=pl.BlockSpec((TILE, DIM), lambda t: (t, 0)),
        out_shape=jax.ShapeDtypeStruct((T, DIM), jnp.float32),
    )(xt, fc1, fc2, fc3)

    g0 = eo[pos0]
    g1 = eo[pos1]

    out = pl.pallas_call(
        _combine_kernel,
        grid=(T // TILE,),
        in_specs=[
            pl.BlockSpec((TILE, DIM), lambda t: (t, 0)),
            pl.BlockSpec((TILE, DIM), lambda t: (t, 0)),
            pl.BlockSpec((TILE, DIM), lambda t: (t, 0)),
            pl.BlockSpec((8, TILE), lambda t: (0, t)),
        ],
        out_specs=pl.BlockSpec((TILE, DIM), lambda t: (t, 0)),
        out_shape=jax.ShapeDtypeStruct((T, DIM), jnp.float32),
    )(z, g0, g1, rout)

    return out.reshape(orig_shape)
